# baseline (speedup 1.0000x reference)
import functools

import jax
import jax.numpy as jnp
from jax import lax
from jax.experimental import pallas as pl
from jax.experimental.pallas import tpu as pltpu
from jax.experimental.pallas import tpu_sc as plsc

_L = 16384
_D = 64
_LANES = 16
_NC = 2
_NS = 16
_NW = _NC * _NS
_BPW = _L // _NW
_WIN = 128
_NWIN = _BPW // _WIN


def _sc_gather_sum(tokens_3d, emb_table):
    mesh = plsc.VectorSubcoreMesh(core_axis_name="c", subcore_axis_name="s")

    @functools.partial(
        pl.kernel,
        out_type=jax.ShapeDtypeStruct((_NW, _D), jnp.float32),
        mesh=mesh,
        compiler_params=pltpu.CompilerParams(use_tc_tiling_on_sc=False),
        scratch_types=[
            pltpu.VMEM((_NWIN, _WIN), jnp.int32),
            pltpu.VMEM((_WIN, _D), jnp.float32),
            pltpu.VMEM((_WIN, _D), jnp.float32),
            pltpu.VMEM((_D,), jnp.float32),
            pltpu.SemaphoreType.DMA,
            pltpu.SemaphoreType.DMA,
        ],
    )
    def sc_kernel(tok_hbm, emb_hbm, out_hbm, idx_v, buf0, buf1, acc_v,
                  sem0, sem1):
        wid = lax.axis_index("s") * _NC + lax.axis_index("c")
        pltpu.sync_copy(tok_hbm.at[wid], idx_v)

        bufs = (buf0, buf1)
        sems = (sem0, sem1)
        copies = [None, None]
        copies[0] = pltpu.async_copy(emb_hbm.at[idx_v.at[0]], buf0, sem0)

        acc = tuple(jnp.zeros((_LANES,), jnp.float32) for _ in range(4))
        for w in range(_NWIN):
            cur = w & 1
            copies[cur].wait()
            if w + 1 < _NWIN:
                copies[1 - cur] = pltpu.async_copy(
                    emb_hbm.at[idx_v.at[w + 1]], bufs[1 - cur], sems[1 - cur])
            buf = bufs[cur]

            def body(r, a, buf=buf):
                return tuple(a[c] + buf[r, pl.ds(c * _LANES, _LANES)]
                             for c in range(4))

            acc = lax.fori_loop(0, _WIN, body, acc)

        for c in range(4):
            acc_v[pl.ds(c * _LANES, _LANES)] = acc[c]
        pltpu.sync_copy(acc_v, out_hbm.at[wid])

    return sc_kernel(tokens_3d, emb_table)


def _tc_combine(partials, W, b_col):
    def body(p_ref, w_ref, b_ref, o_ref):
        enc = jnp.sum(p_ref[...], axis=0, keepdims=True) * (1.0 / _L)
        prod = w_ref[...] * enc
        o_ref[...] = jnp.sum(prod, axis=1, keepdims=True) + b_ref[...]

    return pl.pallas_call(
        body,
        out_shape=jax.ShapeDtypeStruct((5, 1), jnp.float32),
    )(partials, W, b_col)


@jax.jit
def kernel(tokens, emb_table, W, b):
    tokens_3d = tokens.astype(jnp.int32).reshape(_NW, _NWIN, _WIN)
    partials = _sc_gather_sum(tokens_3d, emb_table)
    logits = _tc_combine(partials, W, b.reshape(5, 1))
    return logits[:, 0]


# speedup vs baseline: 1.6854x; 1.6854x over previous
import functools

import jax
import jax.numpy as jnp
from jax import lax
from jax.experimental import pallas as pl
from jax.experimental.pallas import tpu as pltpu
from jax.experimental.pallas import tpu_sc as plsc

_L = 16384
_D = 64
_LANES = 16
_NC = 2
_NS = 16
_NW = _NC * _NS
_BPW = _L // _NW
_CH = _BPW // _LANES


def _sc_gather_sum(tokens_2d, emb_table):
    mesh = plsc.VectorSubcoreMesh(core_axis_name="c", subcore_axis_name="s")

    @functools.partial(
        pl.kernel,
        out_type=jax.ShapeDtypeStruct((_NW, _D), jnp.float32),
        mesh=mesh,
        scratch_types=[
            pltpu.VMEM((_BPW,), jnp.int32),
            pltpu.VMEM((2 * _LANES, _D), jnp.float32),
            pltpu.VMEM((_D,), jnp.float32),
            pltpu.SemaphoreType.DMA,
            pltpu.SemaphoreType.DMA,
        ],
    )
    def sc_kernel(tok_hbm, emb_hbm, out_hbm, idx_v, ring, acc_v, sem_a, sem_b):
        wid = lax.axis_index("s") * _NC + lax.axis_index("c")
        pltpu.sync_copy(tok_hbm.at[wid], idx_v)

        def fire(chunk, bank_base, sem):
            cv = idx_v[pl.ds(chunk * _LANES, _LANES)]
            for l in range(_LANES):
                pltpu.async_copy(emb_hbm.at[pl.ds(cv[l], 1)],
                                 ring.at[pl.ds(bank_base + l, 1)], sem)

        def drain(bank_base, sem):
            pltpu.make_async_copy(emb_hbm.at[pl.ds(0, _LANES)],
                                  ring.at[pl.ds(bank_base, _LANES)], sem).wait()

        fire(0, 0, sem_a)
        fire(1, _LANES, sem_b)

        def do_bank(acc, chunk, bank_base, sem):
            drain(bank_base, sem)
            rows = [tuple(ring[bank_base + l, pl.ds(c * _LANES, _LANES)]
                          for c in range(4)) for l in range(_LANES)]

            @pl.when(chunk + 2 < _CH)
            def _():
                fire(chunk + 2, bank_base, sem)

            return tuple(acc[c] + sum(r[c] for r in rows) for c in range(4))

        def body(i, acc):
            acc = do_bank(acc, 2 * i, 0, sem_a)
            acc = do_bank(acc, 2 * i + 1, _LANES, sem_b)
            return acc

        acc0 = tuple(jnp.zeros((_LANES,), jnp.float32) for _ in range(4))
        acc = lax.fori_loop(0, _CH // 2, body, acc0)

        for c in range(4):
            acc_v[pl.ds(c * _LANES, _LANES)] = acc[c]
        pltpu.sync_copy(acc_v, out_hbm.at[wid])

    return sc_kernel(tokens_2d, emb_table)


def _tc_combine(partials, W, b_col):
    def body(p_ref, w_ref, b_ref, o_ref):
        enc = jnp.sum(p_ref[...], axis=0, keepdims=True) * (1.0 / _L)
        prod = w_ref[...] * enc
        o_ref[...] = jnp.sum(prod, axis=1, keepdims=True) + b_ref[...]

    return pl.pallas_call(
        body,
        out_shape=jax.ShapeDtypeStruct((5, 1), jnp.float32),
    )(partials, W, b_col)


@jax.jit
def kernel(tokens, emb_table, W, b):
    tokens_2d = tokens.astype(jnp.int32).reshape(_NW, _BPW)
    partials = _sc_gather_sum(tokens_2d, emb_table)
    logits = _tc_combine(partials, W, b.reshape(5, 1))
    return logits[:, 0]


# speedup vs baseline: 5.3862x; 3.1958x over previous
import functools

import jax
import jax.numpy as jnp
from jax import lax
from jax.experimental import pallas as pl
from jax.experimental.pallas import tpu as pltpu
from jax.experimental.pallas import tpu_sc as plsc

_V = 1_000_000
_L = 16384
_D = 64
_LANES = 16
_NC = 2
_NS = 16
_BPW = _L // (_NC * _NS)
_BLK = 32_768


def _sc_counts(tokens_4d, zeros_hbm):
    mesh = plsc.VectorSubcoreMesh(core_axis_name="c", subcore_axis_name="s")

    @functools.partial(
        pl.kernel,
        out_type=[jax.ShapeDtypeStruct((_V,), jnp.float32),
                  jax.ShapeDtypeStruct((_V,), jnp.float32)],
        mesh=mesh,
        scratch_types=[
            pltpu.VMEM((1, _BPW), jnp.int32),
            pltpu.VMEM((1, _BPW), jnp.float32),
            pltpu.VMEM_SHARED((_V,), jnp.float32),
        ],
    )
    def sc_kernel(tok_hbm, zeros_hbm_ref, out0, out1, idx_v, ones_v,
                  counts_sp):
        c = lax.axis_index("c")
        s = lax.axis_index("s")

        @pl.when(s == 0)
        def _():
            pltpu.sync_copy(zeros_hbm_ref, counts_sp)

        for i in range(0, _BPW, _LANES):
            ones_v[0, pl.ds(i, _LANES)] = jnp.ones((_LANES,), jnp.float32)
        pltpu.sync_copy(tok_hbm.at[c, s], idx_v)

        plsc.subcore_barrier()
        pltpu.sync_copy(ones_v.at[0], counts_sp.at[idx_v.at[0]], add=True)
        plsc.subcore_barrier()

        @pl.when(jnp.logical_and(s == 0, c == 0))
        def _():
            pltpu.sync_copy(counts_sp, out0)

        @pl.when(jnp.logical_and(s == 0, c == 1))
        def _():
            pltpu.sync_copy(counts_sp, out1)

    return sc_kernel(tokens_4d, zeros_hbm)


def _tc_logits(table_t, counts0, counts1, W, b_col):
    def body(p_ref, c0_ref, c1_ref, w_ref, b_ref, o_ref, acc_ref):
        i = pl.program_id(0)

        @pl.when(i == 0)
        def _():
            acc_ref[...] = jnp.zeros_like(acc_ref)

        cb = (c0_ref[...] + c1_ref[...])[None, :]
        lane = jax.lax.broadcasted_iota(jnp.int32, (1, _BLK), 1)
        valid = lane < (_V - i * _BLK)
        prod = jnp.where(valid, p_ref[...] * cb, 0.0)
        acc_ref[...] += jnp.sum(prod, axis=1, keepdims=True)

        @pl.when(i == pl.num_programs(0) - 1)
        def _():
            enc = acc_ref[...] * (1.0 / _L)
            o_ref[...] = jnp.dot(
                w_ref[...], enc,
                preferred_element_type=jnp.float32) + b_ref[...]

    return pl.pallas_call(
        body,
        grid=(pl.cdiv(_V, _BLK),),
        in_specs=[
            pl.BlockSpec((_D, _BLK), lambda i: (0, i)),
            pl.BlockSpec((_BLK,), lambda i: (i,)),
            pl.BlockSpec((_BLK,), lambda i: (i,)),
            pl.BlockSpec((5, _D), lambda i: (0, 0)),
            pl.BlockSpec((5, 1), lambda i: (0, 0)),
        ],
        out_specs=pl.BlockSpec((5, 1), lambda i: (0, 0)),
        out_shape=jax.ShapeDtypeStruct((5, 1), jnp.float32),
        scratch_shapes=[pltpu.VMEM((_D, 1), jnp.float32)],
    )(table_t, counts0, counts1, W, b_col)


@jax.jit
def kernel(tokens, emb_table, W, b):
    tokens_4d = tokens.astype(jnp.int32).reshape(_NC, _NS, 1, _BPW)
    counts0, counts1 = _sc_counts(tokens_4d, jnp.zeros((_V,), jnp.float32))
    logits = _tc_logits(emb_table.T, counts0, counts1, W, b.reshape(5, 1))
    return logits[:, 0]


# speedup vs baseline: 5.4532x; 1.0124x over previous
import functools

import jax
import jax.numpy as jnp
from jax import lax
from jax.experimental import pallas as pl
from jax.experimental.pallas import tpu as pltpu
from jax.experimental.pallas import tpu_sc as plsc

_V = 1_000_000
_L = 16384
_D = 64
_LANES = 16
_NC = 2
_NS = 16
_BPW = _L // (_NC * _NS)
_BLK = 65_536


def _sc_counts(tokens_4d, zeros_hbm):
    mesh = plsc.VectorSubcoreMesh(core_axis_name="c", subcore_axis_name="s")

    @functools.partial(
        pl.kernel,
        out_type=[jax.ShapeDtypeStruct((_V,), jnp.float32),
                  jax.ShapeDtypeStruct((_V,), jnp.float32)],
        mesh=mesh,
        scratch_types=[
            pltpu.VMEM((1, _BPW), jnp.int32),
            pltpu.VMEM((1, _BPW), jnp.float32),
            pltpu.VMEM_SHARED((_V,), jnp.float32),
        ],
    )
    def sc_kernel(tok_hbm, zeros_hbm_ref, out0, out1, idx_v, ones_v,
                  counts_sp):
        c = lax.axis_index("c")
        s = lax.axis_index("s")

        @pl.when(s == 0)
        def _():
            pltpu.sync_copy(zeros_hbm_ref, counts_sp)

        for i in range(0, _BPW, _LANES):
            ones_v[0, pl.ds(i, _LANES)] = jnp.ones((_LANES,), jnp.float32)
        pltpu.sync_copy(tok_hbm.at[c, s], idx_v)

        plsc.subcore_barrier()
        pltpu.sync_copy(ones_v.at[0], counts_sp.at[idx_v.at[0]], add=True)
        plsc.subcore_barrier()

        @pl.when(jnp.logical_and(s == 0, c == 0))
        def _():
            pltpu.sync_copy(counts_sp, out0)

        @pl.when(jnp.logical_and(s == 0, c == 1))
        def _():
            pltpu.sync_copy(counts_sp, out1)

    return sc_kernel(tokens_4d, zeros_hbm)


def _tc_logits(table_t, counts0, counts1, W, b_col):
    def body(p_ref, c0_ref, c1_ref, w_ref, b_ref, o_ref, acc_ref):
        i = pl.program_id(0)

        @pl.when(i == 0)
        def _():
            acc_ref[...] = jnp.zeros_like(acc_ref)

        cb = (c0_ref[...] + c1_ref[...])[None, :]
        last = pl.num_programs(0) - 1

        @pl.when(i < last)
        def _():
            prod = p_ref[...] * cb
            acc_ref[...] += jnp.sum(prod, axis=1, keepdims=True)

        @pl.when(i == last)
        def _():
            lane = jax.lax.broadcasted_iota(jnp.int32, (1, _BLK), 1)
            valid = lane < (_V - i * _BLK)
            prod = jnp.where(valid, p_ref[...] * cb, 0.0)
            acc_ref[...] += jnp.sum(prod, axis=1, keepdims=True)

        @pl.when(i == pl.num_programs(0) - 1)
        def _():
            enc = acc_ref[...] * (1.0 / _L)
            o_ref[...] = jnp.dot(
                w_ref[...], enc,
                preferred_element_type=jnp.float32) + b_ref[...]

    return pl.pallas_call(
        body,
        grid=(pl.cdiv(_V, _BLK),),
        in_specs=[
            pl.BlockSpec((_D, _BLK), lambda i: (0, i)),
            pl.BlockSpec((_BLK,), lambda i: (i,)),
            pl.BlockSpec((_BLK,), lambda i: (i,)),
            pl.BlockSpec((5, _D), lambda i: (0, 0)),
            pl.BlockSpec((5, 1), lambda i: (0, 0)),
        ],
        out_specs=pl.BlockSpec((5, 1), lambda i: (0, 0)),
        out_shape=jax.ShapeDtypeStruct((5, 1), jnp.float32),
        scratch_shapes=[pltpu.VMEM((_D, 1), jnp.float32)],
    )(table_t, counts0, counts1, W, b_col)


@jax.jit
def kernel(tokens, emb_table, W, b):
    tokens_4d = tokens.astype(jnp.int32).reshape(_NC, _NS, 1, _BPW)
    counts0, counts1 = _sc_counts(tokens_4d, jnp.zeros((_V,), jnp.float32))
    logits = _tc_logits(emb_table.T, counts0, counts1, W, b.reshape(5, 1))
    return logits[:, 0]
